# baseline (speedup 1.0000x reference)
import functools

import jax
import jax.numpy as jnp
import numpy as np
from jax import lax
from jax.experimental import pallas as pl
from jax.experimental.pallas import tpu as pltpu
from jax.experimental.pallas import tpu_sc as plsc

N = 10000
E = 320000
D_FEAT = 128
D_GLOBAL = 16
D_EDGE = 4
D_EMB = 128
N_CLASSES = 8
SIGMA2 = 1.0 + 1e-06
_I0 = np.int32(0)

EP32 = E // 32
BR = 1000
NB_P = EP32 // BR
WREP = 16
ER8 = E // 8
BR8 = 4000
NB_R = ER8 // BR8


def _min_body(ap_ref, out_ref, mn_ref):
    j = pl.program_id(0)

    @pl.when(j == 0)
    def _():
        mn_ref[...] = jnp.full((1, 128), jnp.inf, jnp.float32)
    mn_ref[...] = jnp.minimum(mn_ref[...],
                              jnp.min(ap_ref[...], axis=0, keepdims=True))
    out_ref[...] = mn_ref[...]


def _lane_min(ap):
    return pl.pallas_call(
        _min_body,
        grid=(NB_P,),
        in_specs=[pl.BlockSpec((BR, 128), lambda j: (j, _I0))],
        out_specs=pl.BlockSpec((1, 128), lambda j: (_I0, _I0)),
        out_shape=jax.ShapeDtypeStruct((1, 128), jnp.float32),
        name="edge_min",
        scratch_shapes=[pltpu.VMEM((1, 128), jnp.float32)],
    )(ap)


def _w32_body(ap_ref, m_ref, we_ref, sel_ref, out_ref):
    p = ap_ref[...] - m_ref[...]
    q = jnp.exp(-(p * p) / SIGMA2) * we_ref[...]
    sv = jnp.dot(q, sel_ref[...], preferred_element_type=jnp.float32)
    out_ref[...] = 1.0 / (1.0 + jnp.exp(-sv))


def _w32(ap, m128, we128, sel):
    return pl.pallas_call(
        _w32_body,
        grid=(NB_P,),
        in_specs=[
            pl.BlockSpec((BR, 128), lambda j: (j, _I0)),
            pl.BlockSpec((1, 128), lambda j: (_I0, _I0)),
            pl.BlockSpec((1, 128), lambda j: (_I0, _I0)),
            pl.BlockSpec((128, 32), lambda j: (_I0, _I0)),
        ],
        out_specs=pl.BlockSpec((BR, 32), lambda j: (j, _I0)),
        out_shape=jax.ShapeDtypeStruct((EP32, 32), jnp.float32),
        name="edge_w32",
    )(ap, m128, we128, sel)


def _rep_body(wf_ref, rr_ref, out_ref):
    out_ref[...] = jnp.dot(wf_ref[...], rr_ref[...],
                           preferred_element_type=jnp.float32)


def _replicate(wf, rr):
    return pl.pallas_call(
        _rep_body,
        grid=(NB_R,),
        in_specs=[
            pl.BlockSpec((BR8, 8), lambda j: (j, _I0)),
            pl.BlockSpec((8, 128), lambda j: (_I0, _I0)),
        ],
        out_specs=pl.BlockSpec((BR8, 128), lambda j: (j, _I0)),
        out_shape=jax.ShapeDtypeStruct((ER8, 128), jnp.float32),
        name="edge_w_rep",
    )(wf, rr)


def _edge_weights(edge_attr, We):
    ap = edge_attr.reshape(EP32, 128)
    lane_min = _lane_min(ap)
    min4 = lane_min.reshape(32, D_EDGE).min(axis=0)
    m128 = jnp.tile(min4, 32).reshape(1, 128)
    we128 = jnp.tile(We.reshape(D_EDGE), 32).reshape(1, 128)
    sel = jnp.repeat(jnp.eye(32, dtype=jnp.float32), D_EDGE, axis=0)
    w32 = _w32(ap, m128, we128, sel)
    rr = jnp.repeat(jnp.eye(8, dtype=jnp.float32), WREP, axis=1)
    wrep = _replicate(w32.reshape(ER8, 8), rr)
    return wrep.reshape(E, WREP)


BN = 1000
NB_H = N // BN


def _h_body(x_ref, w1_ref, out_ref):
    out_ref[...] = jnp.dot(x_ref[...], w1_ref[...],
                           preferred_element_type=jnp.float32)


def _node_emb(x_real, W1):
    return pl.pallas_call(
        _h_body,
        grid=(NB_H,),
        in_specs=[
            pl.BlockSpec((BN, D_FEAT), lambda j: (j, _I0)),
            pl.BlockSpec((D_FEAT, D_EMB), lambda j: (_I0, _I0)),
        ],
        out_specs=pl.BlockSpec((BN, D_EMB), lambda j: (j, _I0)),
        out_shape=jax.ShapeDtypeStruct((N, D_EMB), jnp.float32),
        name="h_mm",
    )(x_real, W1)


NTILES = 32
EPT = E // NTILES
C = 128
K = 80
EPT_PAD = K * C
PAIRS = K // 2
RPT = N // 16


def _sc_body(h_hbm, src_hbm, dst_hbm, w_hbm, out_hbm,
             src_v, dst0, dst1, w0, w1, rows0, rows1, agg_sh,
             gsem0, gsem1):
    cid = lax.axis_index("c")
    sid = lax.axis_index("s")
    wid = cid * 16 + sid

    def _zb(i, carry):
        for k in range(8):
            rows0[i, pl.ds(k * 16, 16)] = jnp.zeros((16,), jnp.float32)
        return carry
    lax.fori_loop(jnp.int32(0), jnp.int32(C), _zb, jnp.int32(0))
    for t in range(4):
        pltpu.sync_copy(rows0, agg_sh.at[pl.ds(sid * RPT + t * C, C)])
    pltpu.sync_copy(rows0.at[pl.ds(0, RPT - 4 * C)],
                    agg_sh.at[pl.ds(sid * RPT + 4 * C, RPT - 4 * C)])
    plsc.subcore_barrier()

    pltpu.sync_copy(src_hbm.at[wid], src_v)

    def _gather(j, rows, dst, w, gsem):
        pltpu.async_copy(h_hbm.at[src_v.at[j]], rows, gsem)
        pltpu.async_copy(dst_hbm.at[wid, j], dst, gsem)
        pltpu.async_copy(w_hbm.at[wid, j], w, gsem)

    def _gather_wait(j, rows, dst, w, gsem):
        pltpu.make_async_copy(h_hbm.at[src_v.at[j]], rows, gsem).wait()
        pltpu.make_async_copy(dst_hbm.at[wid, j], dst, gsem).wait()
        pltpu.make_async_copy(w_hbm.at[wid, j], w, gsem).wait()

    def _scale(rows, w):
        def body(e, c2):
            off = pl.multiple_of(e * WREP, WREP)
            wj = w[pl.ds(off, WREP)]
            for k in range(8):
                sl = pl.ds(k * 16, 16)
                rows[e, sl] = rows[e, sl] * wj
            return c2
        lax.fori_loop(jnp.int32(0), jnp.int32(C), body, jnp.int32(0))

    _gather(jnp.int32(0), rows0, dst0, w0, gsem0)

    def _pair(m, carry):
        a = 2 * m
        b = a + 1

        _gather(b, rows1, dst1, w1, gsem1)
        _gather_wait(a, rows0, dst0, w0, gsem0)
        _scale(rows0, w0)
        pltpu.sync_copy(rows0, agg_sh.at[dst0], add=True)

        @pl.when(m < PAIRS - 1)
        def _():
            _gather(a + 2, rows0, dst0, w0, gsem0)
        _gather_wait(b, rows1, dst1, w1, gsem1)
        _scale(rows1, w1)
        pltpu.sync_copy(rows1, agg_sh.at[dst1], add=True)
        return carry
    lax.fori_loop(jnp.int32(0), jnp.int32(PAIRS), _pair, jnp.int32(0))
    plsc.subcore_barrier()

    pltpu.sync_copy(agg_sh.at[pl.ds(sid * RPT, RPT)], out_hbm.at[wid])


def _sc_scatter(h, src3d, dst3d, w4d):
    mesh = plsc.VectorSubcoreMesh(core_axis_name="c", subcore_axis_name="s")
    fn = functools.partial(
        pl.kernel,
        out_type=jax.ShapeDtypeStruct((NTILES, RPT, D_EMB), jnp.float32),
        mesh=mesh,
        name="sc_scatter",
        scratch_types=[
            pltpu.VMEM((K, C), jnp.int32),
            pltpu.VMEM((C,), jnp.int32),
            pltpu.VMEM((C,), jnp.int32),
            pltpu.VMEM((C * WREP,), jnp.float32),
            pltpu.VMEM((C * WREP,), jnp.float32),
            pltpu.VMEM((C, D_EMB), jnp.float32),
            pltpu.VMEM((C, D_EMB), jnp.float32),
            pltpu.VMEM_SHARED((N, D_EMB), jnp.float32),
            pltpu.SemaphoreType.DMA,
            pltpu.SemaphoreType.DMA,
        ],
    )(_sc_body)
    return fn(h, src3d, dst3d, w4d)


def _final_body(p0_ref, p1_ref, w2_ref, b2_ref, wc_ref, bc_ref, g_ref,
                out_ref, acc_ref):
    j = pl.program_id(0)

    @pl.when(j == 0)
    def _():
        acc_ref[...] = jnp.zeros_like(acc_ref)
        out_ref[...] = jnp.zeros_like(out_ref)

    s = p0_ref[...] + p1_ref[...]
    t = jnp.dot(s, w2_ref[...], preferred_element_type=jnp.float32)
    t = jnp.maximum(t + b2_ref[...], 0.0)
    acc_ref[...] += jnp.sum(t, axis=0, keepdims=True)

    @pl.when(j == NB_H - 1)
    def _():
        o = jnp.dot(acc_ref[...], wc_ref[pl.ds(0, D_EMB), :],
                    preferred_element_type=jnp.float32)
        o += jnp.dot(g_ref[...], wc_ref[pl.ds(D_EMB, D_GLOBAL), :],
                     preferred_element_type=jnp.float32)
        out_ref[...] = o + bc_ref[...]


def _final(parts, W2, b2r, Wc, bcr, g):
    return pl.pallas_call(
        _final_body,
        grid=(NB_H,),
        in_specs=[
            pl.BlockSpec((BN, D_EMB), lambda j: (j, _I0)),
            pl.BlockSpec((BN, D_EMB), lambda j: (j + NB_H, _I0)),
            pl.BlockSpec((D_EMB, D_EMB), lambda j: (_I0, _I0)),
            pl.BlockSpec((1, D_EMB), lambda j: (_I0, _I0)),
            pl.BlockSpec((D_EMB + D_GLOBAL, N_CLASSES), lambda j: (_I0, _I0)),
            pl.BlockSpec((1, N_CLASSES), lambda j: (_I0, _I0)),
            pl.BlockSpec((1, D_GLOBAL), lambda j: (_I0, _I0)),
        ],
        out_specs=pl.BlockSpec((1, N_CLASSES), lambda j: (_I0, _I0)),
        out_shape=jax.ShapeDtypeStruct((1, N_CLASSES), jnp.float32),
        name="final",
        scratch_shapes=[pltpu.VMEM((1, D_EMB), jnp.float32)],
    )(parts, parts, W2, b2r, Wc, bcr, g)


def kernel(x, edge_index, edge_attr, batch, W1, We, W2, b2, Wc, bc):
    pad = EPT_PAD - EPT
    src = edge_index[0].astype(jnp.int32).reshape(NTILES, EPT)
    dst = edge_index[1].astype(jnp.int32).reshape(NTILES, EPT)
    src3d = jnp.pad(src, ((0, 0), (0, pad))).reshape(NTILES, K, C)
    dst3d = jnp.pad(dst, ((0, 0), (0, pad))).reshape(NTILES, K, C)
    x_real = x[:, :D_FEAT]
    g = x[0:1, D_FEAT:]

    w_e = _edge_weights(edge_attr, We)
    w4d = jnp.pad(w_e.reshape(NTILES, EPT, WREP),
                  ((0, 0), (0, EPT_PAD - EPT), (0, 0))
                  ).reshape(NTILES, K, C * WREP)
    h = _node_emb(x_real, W1)
    parts = _sc_scatter(h, src3d, dst3d, w4d)
    parts = parts.reshape(2 * N, D_EMB)
    out = _final(parts, W2, b2.reshape(1, D_EMB), Wc,
                 bc.reshape(1, N_CLASSES), g)
    return out


# speedup vs baseline: 1.3643x; 1.3643x over previous
import functools

import jax
import jax.numpy as jnp
import numpy as np
from jax import lax
from jax.experimental import pallas as pl
from jax.experimental.pallas import tpu as pltpu
from jax.experimental.pallas import tpu_sc as plsc

N = 10000
E = 320000
D_FEAT = 128
D_GLOBAL = 16
D_EDGE = 4
D_EMB = 128
N_CLASSES = 8
SIGMA2 = 1.0 + 1e-06
_I0 = np.int32(0)

EP32 = E // 32
BR = 1000
NB_P = EP32 // BR
WREP = 16
ER8 = E // 8
BR8 = 4000
NB_R = ER8 // BR8


def _min_body(ap_ref, out_ref, mn_ref):
    j = pl.program_id(0)

    @pl.when(j == 0)
    def _():
        mn_ref[...] = jnp.full((1, 128), jnp.inf, jnp.float32)
    mn_ref[...] = jnp.minimum(mn_ref[...],
                              jnp.min(ap_ref[...], axis=0, keepdims=True))
    out_ref[...] = mn_ref[...]


def _lane_min(ap):
    return pl.pallas_call(
        _min_body,
        grid=(NB_P,),
        in_specs=[pl.BlockSpec((BR, 128), lambda j: (j, _I0))],
        out_specs=pl.BlockSpec((1, 128), lambda j: (_I0, _I0)),
        out_shape=jax.ShapeDtypeStruct((1, 128), jnp.float32),
        name="edge_min",
        scratch_shapes=[pltpu.VMEM((1, 128), jnp.float32)],
    )(ap)


def _w32_body(ap_ref, m_ref, we_ref, sel_ref, out_ref):
    p = ap_ref[...] - m_ref[...]
    q = jnp.exp(-(p * p) / SIGMA2) * we_ref[...]
    sv = jnp.dot(q, sel_ref[...], preferred_element_type=jnp.float32)
    out_ref[...] = 1.0 / (1.0 + jnp.exp(-sv))


def _w32(ap, m128, we128, sel):
    return pl.pallas_call(
        _w32_body,
        grid=(NB_P,),
        in_specs=[
            pl.BlockSpec((BR, 128), lambda j: (j, _I0)),
            pl.BlockSpec((1, 128), lambda j: (_I0, _I0)),
            pl.BlockSpec((1, 128), lambda j: (_I0, _I0)),
            pl.BlockSpec((128, 32), lambda j: (_I0, _I0)),
        ],
        out_specs=pl.BlockSpec((BR, 32), lambda j: (j, _I0)),
        out_shape=jax.ShapeDtypeStruct((EP32, 32), jnp.float32),
        name="edge_w32",
    )(ap, m128, we128, sel)


def _rep_body(wf_ref, rr_ref, out_ref):
    out_ref[...] = jnp.dot(wf_ref[...], rr_ref[...],
                           preferred_element_type=jnp.float32)


def _replicate(wf, rr):
    return pl.pallas_call(
        _rep_body,
        grid=(NB_R,),
        in_specs=[
            pl.BlockSpec((BR8, 8), lambda j: (j, _I0)),
            pl.BlockSpec((8, 128), lambda j: (_I0, _I0)),
        ],
        out_specs=pl.BlockSpec((BR8, 128), lambda j: (j, _I0)),
        out_shape=jax.ShapeDtypeStruct((ER8, 128), jnp.float32),
        name="edge_w_rep",
    )(wf, rr)


def _edge_weights(edge_attr, We):
    ap = edge_attr.reshape(EP32, 128)
    lane_min = _lane_min(ap)
    min4 = lane_min.reshape(32, D_EDGE).min(axis=0)
    m128 = jnp.tile(min4, 32).reshape(1, 128)
    we128 = jnp.tile(We.reshape(D_EDGE), 32).reshape(1, 128)
    sel = jnp.repeat(jnp.eye(32, dtype=jnp.float32), D_EDGE, axis=0)
    w32 = _w32(ap, m128, we128, sel)
    rr = jnp.repeat(jnp.eye(8, dtype=jnp.float32), WREP, axis=1)
    wrep = _replicate(w32.reshape(ER8, 8), rr)
    return wrep.reshape(E, WREP)


BN = 1000
NB_H = N // BN


def _h_body(x_ref, w1_ref, out_ref):
    out_ref[...] = jnp.dot(x_ref[...], w1_ref[...],
                           preferred_element_type=jnp.float32)


def _node_emb(x_real, W1):
    return pl.pallas_call(
        _h_body,
        grid=(NB_H,),
        in_specs=[
            pl.BlockSpec((BN, D_FEAT), lambda j: (j, _I0)),
            pl.BlockSpec((D_FEAT, D_EMB), lambda j: (_I0, _I0)),
        ],
        out_specs=pl.BlockSpec((BN, D_EMB), lambda j: (j, _I0)),
        out_shape=jax.ShapeDtypeStruct((N, D_EMB), jnp.float32),
        name="h_mm",
    )(x_real, W1)


NTILES = 32
EPT = E // NTILES
C = 125
K = EPT // C
G = 16
NG = K // G
RPT = N // 16


def _sc_body(h_hbm, src_hbm, dst_hbm, w_hbm, out_hbm,
             src_v, dst_v, w_v, rows_v, agg_sh, sem):
    cid = lax.axis_index("c")
    sid = lax.axis_index("s")
    wid = cid * 16 + sid

    def _zb(i, carry):
        for k in range(8):
            rows_v[i, pl.ds(k * 16, 16)] = jnp.zeros((16,), jnp.float32)
        return carry
    lax.fori_loop(jnp.int32(0), jnp.int32(C), _zb, jnp.int32(0))
    for t in range(5):
        pltpu.sync_copy(rows_v, agg_sh.at[pl.ds(sid * RPT + t * C, C)])
    plsc.subcore_barrier()

    for g in range(NG):
        pltpu.sync_copy(src_hbm.at[wid, jnp.int32(g)], src_v)
        pltpu.sync_copy(dst_hbm.at[wid, jnp.int32(g)], dst_v)

        def _chunk(j, carry):
            pltpu.sync_copy(w_hbm.at[wid, jnp.int32(g * G) + j], w_v)
            pltpu.async_copy(h_hbm.at[src_v.at[j]], rows_v, sem).wait()

            def _scale(e, c2):
                wj = w_v[e, :]
                for k in range(8):
                    sl = pl.ds(k * 16, 16)
                    rows_v[e, sl] = rows_v[e, sl] * wj
                return c2
            lax.fori_loop(jnp.int32(0), jnp.int32(C), _scale, jnp.int32(0))

            pltpu.sync_copy(rows_v, agg_sh.at[dst_v.at[j]], add=True)
            return carry
        lax.fori_loop(jnp.int32(0), jnp.int32(G), _chunk, jnp.int32(0))
    plsc.subcore_barrier()

    pltpu.sync_copy(agg_sh.at[pl.ds(sid * RPT, RPT)], out_hbm.at[wid])


def _sc_scatter(h, src3d, dst3d, w4d):
    mesh = plsc.VectorSubcoreMesh(core_axis_name="c", subcore_axis_name="s")
    fn = functools.partial(
        pl.kernel,
        out_type=jax.ShapeDtypeStruct((NTILES, RPT, D_EMB), jnp.float32),
        mesh=mesh,
        name="sc_scatter",
        scratch_types=[
            pltpu.VMEM((G, C), jnp.int32),
            pltpu.VMEM((G, C), jnp.int32),
            pltpu.VMEM((C, WREP), jnp.float32),
            pltpu.VMEM((C, D_EMB), jnp.float32),
            pltpu.VMEM_SHARED((N, D_EMB), jnp.float32),
            pltpu.SemaphoreType.DMA,
        ],
    )(_sc_body)
    return fn(h, src3d, dst3d, w4d)


def _final_body(p0_ref, p1_ref, w2_ref, b2_ref, wc_ref, bc_ref, g_ref,
                out_ref, acc_ref):
    j = pl.program_id(0)

    @pl.when(j == 0)
    def _():
        acc_ref[...] = jnp.zeros_like(acc_ref)
        out_ref[...] = jnp.zeros_like(out_ref)

    s = p0_ref[...] + p1_ref[...]
    t = jnp.dot(s, w2_ref[...], preferred_element_type=jnp.float32)
    t = jnp.maximum(t + b2_ref[...], 0.0)
    acc_ref[...] += jnp.sum(t, axis=0, keepdims=True)

    @pl.when(j == NB_H - 1)
    def _():
        o = jnp.dot(acc_ref[...], wc_ref[pl.ds(0, D_EMB), :],
                    preferred_element_type=jnp.float32)
        o += jnp.dot(g_ref[...], wc_ref[pl.ds(D_EMB, D_GLOBAL), :],
                     preferred_element_type=jnp.float32)
        out_ref[...] = o + bc_ref[...]


def _final(parts, W2, b2r, Wc, bcr, g):
    return pl.pallas_call(
        _final_body,
        grid=(NB_H,),
        in_specs=[
            pl.BlockSpec((BN, D_EMB), lambda j: (j, _I0)),
            pl.BlockSpec((BN, D_EMB), lambda j: (j + NB_H, _I0)),
            pl.BlockSpec((D_EMB, D_EMB), lambda j: (_I0, _I0)),
            pl.BlockSpec((1, D_EMB), lambda j: (_I0, _I0)),
            pl.BlockSpec((D_EMB + D_GLOBAL, N_CLASSES), lambda j: (_I0, _I0)),
            pl.BlockSpec((1, N_CLASSES), lambda j: (_I0, _I0)),
            pl.BlockSpec((1, D_GLOBAL), lambda j: (_I0, _I0)),
        ],
        out_specs=pl.BlockSpec((1, N_CLASSES), lambda j: (_I0, _I0)),
        out_shape=jax.ShapeDtypeStruct((1, N_CLASSES), jnp.float32),
        name="final",
        scratch_shapes=[pltpu.VMEM((1, D_EMB), jnp.float32)],
    )(parts, parts, W2, b2r, Wc, bcr, g)


def kernel(x, edge_index, edge_attr, batch, W1, We, W2, b2, Wc, bc):
    src = edge_index[0].astype(jnp.int32)
    dst = edge_index[1].astype(jnp.int32)
    src3d = src.reshape(NTILES, NG, G, C)
    dst3d = dst.reshape(NTILES, NG, G, C)
    x_real = x[:, :D_FEAT]
    g = x[0:1, D_FEAT:]

    w_e = _edge_weights(edge_attr, We)
    w4d = w_e.reshape(NTILES, K, C, WREP)
    h = _node_emb(x_real, W1)
    parts = _sc_scatter(h, src3d, dst3d, w4d)
    parts = parts.reshape(2 * N, D_EMB)
    out = _final(parts, W2, b2.reshape(1, D_EMB), Wc,
                 bc.reshape(1, N_CLASSES), g)
    return out


# speedup vs baseline: 1.4652x; 1.0740x over previous
import functools

import jax
import jax.numpy as jnp
import numpy as np
from jax import lax
from jax.experimental import pallas as pl
from jax.experimental.pallas import tpu as pltpu
from jax.experimental.pallas import tpu_sc as plsc

N = 10000
E = 320000
D_FEAT = 128
D_GLOBAL = 16
D_EDGE = 4
D_EMB = 128
N_CLASSES = 8
SIGMA2 = 1.0 + 1e-06
_I0 = np.int32(0)

EP32 = E // 32
BR = 1000
NB_P = EP32 // BR
WREP = 16
ER8 = E // 8
BR8 = 4000
NB_R = ER8 // BR8


def _min_body(ap_ref, out_ref, mn_ref):
    j = pl.program_id(0)

    @pl.when(j == 0)
    def _():
        mn_ref[...] = jnp.full((1, 128), jnp.inf, jnp.float32)
    mn_ref[...] = jnp.minimum(mn_ref[...],
                              jnp.min(ap_ref[...], axis=0, keepdims=True))
    out_ref[...] = mn_ref[...]


def _lane_min(ap):
    return pl.pallas_call(
        _min_body,
        grid=(NB_P,),
        in_specs=[pl.BlockSpec((BR, 128), lambda j: (j, _I0))],
        out_specs=pl.BlockSpec((1, 128), lambda j: (_I0, _I0)),
        out_shape=jax.ShapeDtypeStruct((1, 128), jnp.float32),
        name="edge_min",
        scratch_shapes=[pltpu.VMEM((1, 128), jnp.float32)],
    )(ap)


def _w32_body(ap_ref, m_ref, we_ref, sel_ref, out_ref):
    p = ap_ref[...] - m_ref[...]
    q = jnp.exp(-(p * p) / SIGMA2) * we_ref[...]
    sv = jnp.dot(q, sel_ref[...], preferred_element_type=jnp.float32)
    out_ref[...] = 1.0 / (1.0 + jnp.exp(-sv))


def _w32(ap, m128, we128, sel):
    return pl.pallas_call(
        _w32_body,
        grid=(NB_P,),
        in_specs=[
            pl.BlockSpec((BR, 128), lambda j: (j, _I0)),
            pl.BlockSpec((1, 128), lambda j: (_I0, _I0)),
            pl.BlockSpec((1, 128), lambda j: (_I0, _I0)),
            pl.BlockSpec((128, 32), lambda j: (_I0, _I0)),
        ],
        out_specs=pl.BlockSpec((BR, 32), lambda j: (j, _I0)),
        out_shape=jax.ShapeDtypeStruct((EP32, 32), jnp.float32),
        name="edge_w32",
    )(ap, m128, we128, sel)


def _rep_body(wf_ref, rr_ref, out_ref):
    out_ref[...] = jnp.dot(wf_ref[...], rr_ref[...],
                           preferred_element_type=jnp.float32)


def _replicate(wf, rr):
    return pl.pallas_call(
        _rep_body,
        grid=(NB_R,),
        in_specs=[
            pl.BlockSpec((BR8, 8), lambda j: (j, _I0)),
            pl.BlockSpec((8, 128), lambda j: (_I0, _I0)),
        ],
        out_specs=pl.BlockSpec((BR8, 128), lambda j: (j, _I0)),
        out_shape=jax.ShapeDtypeStruct((ER8, 128), jnp.float32),
        name="edge_w_rep",
    )(wf, rr)


def _edge_weights(edge_attr, We):
    ap = edge_attr.reshape(EP32, 128)
    lane_min = _lane_min(ap)
    min4 = lane_min.reshape(32, D_EDGE).min(axis=0)
    m128 = jnp.tile(min4, 32).reshape(1, 128)
    we128 = jnp.tile(We.reshape(D_EDGE), 32).reshape(1, 128)
    sel = jnp.repeat(jnp.eye(32, dtype=jnp.float32), D_EDGE, axis=0)
    w32 = _w32(ap, m128, we128, sel)
    rr = jnp.repeat(jnp.eye(8, dtype=jnp.float32), WREP, axis=1)
    wrep = _replicate(w32.reshape(ER8, 8), rr)
    return wrep.reshape(E, WREP)


BN = 1000
NB_H = N // BN


def _h_body(x_ref, w1_ref, out_ref):
    out_ref[...] = jnp.dot(x_ref[...], w1_ref[...],
                           preferred_element_type=jnp.float32)


def _node_emb(x_real, W1):
    return pl.pallas_call(
        _h_body,
        grid=(NB_H,),
        in_specs=[
            pl.BlockSpec((BN, D_FEAT), lambda j: (j, _I0)),
            pl.BlockSpec((D_FEAT, D_EMB), lambda j: (_I0, _I0)),
        ],
        out_specs=pl.BlockSpec((BN, D_EMB), lambda j: (j, _I0)),
        out_shape=jax.ShapeDtypeStruct((N, D_EMB), jnp.float32),
        name="h_mm",
    )(x_real, W1)


NTILES = 32
EPT = E // NTILES
C = 125
K = EPT // C
G = 16
NG = K // G
RPT = N // 16


def _sc_body(h_hbm, src_hbm, dst_hbm, w_hbm, out_hbm,
             src_v, dst_v, w_v, rows_v, agg_sh, sem):
    cid = lax.axis_index("c")
    sid = lax.axis_index("s")
    wid = cid * 16 + sid

    def _zb(i, carry):
        for k in range(8):
            rows_v[i, pl.ds(k * 16, 16)] = jnp.zeros((16,), jnp.float32)
        return carry
    lax.fori_loop(jnp.int32(0), jnp.int32(C), _zb, jnp.int32(0))
    for t in range(5):
        pltpu.sync_copy(rows_v, agg_sh.at[pl.ds(sid * RPT + t * C, C)])
    plsc.subcore_barrier()

    for g in range(NG):
        pltpu.sync_copy(src_hbm.at[wid, jnp.int32(g)], src_v)
        pltpu.sync_copy(dst_hbm.at[wid, jnp.int32(g)], dst_v)

        def _chunk(j, carry):
            dw = pltpu.async_copy(w_hbm.at[wid, jnp.int32(g * G) + j],
                                  w_v, sem)
            dr = pltpu.async_copy(h_hbm.at[src_v.at[j]], rows_v, sem)
            dw.wait()
            dr.wait()

            def _scale(e, c2):
                wj = w_v[e, :]
                for k in range(8):
                    sl = pl.ds(k * 16, 16)
                    rows_v[e, sl] = rows_v[e, sl] * wj
                return c2
            lax.fori_loop(jnp.int32(0), jnp.int32(C), _scale, jnp.int32(0))

            pltpu.sync_copy(rows_v, agg_sh.at[dst_v.at[j]], add=True)
            return carry
        lax.fori_loop(jnp.int32(0), jnp.int32(G), _chunk, jnp.int32(0))
    plsc.subcore_barrier()

    pltpu.sync_copy(agg_sh.at[pl.ds(sid * RPT, RPT)], out_hbm.at[wid])


def _sc_scatter(h, src3d, dst3d, w4d):
    mesh = plsc.VectorSubcoreMesh(core_axis_name="c", subcore_axis_name="s")
    fn = functools.partial(
        pl.kernel,
        out_type=jax.ShapeDtypeStruct((NTILES, RPT, D_EMB), jnp.float32),
        mesh=mesh,
        name="sc_scatter",
        scratch_types=[
            pltpu.VMEM((G, C), jnp.int32),
            pltpu.VMEM((G, C), jnp.int32),
            pltpu.VMEM((C, WREP), jnp.float32),
            pltpu.VMEM((C, D_EMB), jnp.float32),
            pltpu.VMEM_SHARED((N, D_EMB), jnp.float32),
            pltpu.SemaphoreType.DMA,
        ],
    )(_sc_body)
    return fn(h, src3d, dst3d, w4d)


def _final_body(p0_ref, p1_ref, w2_ref, b2_ref, wc_ref, bc_ref, g_ref,
                out_ref, acc_ref):
    j = pl.program_id(0)

    @pl.when(j == 0)
    def _():
        acc_ref[...] = jnp.zeros_like(acc_ref)
        out_ref[...] = jnp.zeros_like(out_ref)

    s = p0_ref[...] + p1_ref[...]
    t = jnp.dot(s, w2_ref[...], preferred_element_type=jnp.float32)
    t = jnp.maximum(t + b2_ref[...], 0.0)
    acc_ref[...] += jnp.sum(t, axis=0, keepdims=True)

    @pl.when(j == NB_H - 1)
    def _():
        o = jnp.dot(acc_ref[...], wc_ref[pl.ds(0, D_EMB), :],
                    preferred_element_type=jnp.float32)
        o += jnp.dot(g_ref[...], wc_ref[pl.ds(D_EMB, D_GLOBAL), :],
                     preferred_element_type=jnp.float32)
        out_ref[...] = o + bc_ref[...]


def _final(parts, W2, b2r, Wc, bcr, g):
    return pl.pallas_call(
        _final_body,
        grid=(NB_H,),
        in_specs=[
            pl.BlockSpec((BN, D_EMB), lambda j: (j, _I0)),
            pl.BlockSpec((BN, D_EMB), lambda j: (j + NB_H, _I0)),
            pl.BlockSpec((D_EMB, D_EMB), lambda j: (_I0, _I0)),
            pl.BlockSpec((1, D_EMB), lambda j: (_I0, _I0)),
            pl.BlockSpec((D_EMB + D_GLOBAL, N_CLASSES), lambda j: (_I0, _I0)),
            pl.BlockSpec((1, N_CLASSES), lambda j: (_I0, _I0)),
            pl.BlockSpec((1, D_GLOBAL), lambda j: (_I0, _I0)),
        ],
        out_specs=pl.BlockSpec((1, N_CLASSES), lambda j: (_I0, _I0)),
        out_shape=jax.ShapeDtypeStruct((1, N_CLASSES), jnp.float32),
        name="final",
        scratch_shapes=[pltpu.VMEM((1, D_EMB), jnp.float32)],
    )(parts, parts, W2, b2r, Wc, bcr, g)


def kernel(x, edge_index, edge_attr, batch, W1, We, W2, b2, Wc, bc):
    src = edge_index[0].astype(jnp.int32)
    dst = edge_index[1].astype(jnp.int32)
    src3d = src.reshape(NTILES, NG, G, C)
    dst3d = dst.reshape(NTILES, NG, G, C)
    x_real = x[:, :D_FEAT]
    g = x[0:1, D_FEAT:]

    w_e = _edge_weights(edge_attr, We)
    w4d = w_e.reshape(NTILES, K, C, WREP)
    h = _node_emb(x_real, W1)
    parts = _sc_scatter(h, src3d, dst3d, w4d)
    parts = parts.reshape(2 * N, D_EMB)
    out = _final(parts, W2, b2.reshape(1, D_EMB), Wc,
                 bc.reshape(1, N_CLASSES), g)
    return out


# speedup vs baseline: 1.8737x; 1.2788x over previous
import functools

import jax
import jax.numpy as jnp
import numpy as np
from jax import lax
from jax.experimental import pallas as pl
from jax.experimental.pallas import tpu as pltpu
from jax.experimental.pallas import tpu_sc as plsc

N = 10000
E = 320000
D_FEAT = 128
D_GLOBAL = 16
D_EDGE = 4
D_EMB = 128
N_CLASSES = 8
SIGMA2 = 1.0 + 1e-06
_I0 = np.int32(0)

EP32 = E // 32
BR = 1000
NB_P = EP32 // BR
WREP = 16
ER8 = E // 8
BR8 = 4000
NB_R = ER8 // BR8


def _min_body(ap_ref, out_ref, mn_ref):
    j = pl.program_id(0)

    @pl.when(j == 0)
    def _():
        mn_ref[...] = jnp.full((1, 128), jnp.inf, jnp.float32)
    mn_ref[...] = jnp.minimum(mn_ref[...],
                              jnp.min(ap_ref[...], axis=0, keepdims=True))
    out_ref[...] = mn_ref[...]


def _lane_min(ap):
    return pl.pallas_call(
        _min_body,
        grid=(NB_P,),
        in_specs=[pl.BlockSpec((BR, 128), lambda j: (j, _I0))],
        out_specs=pl.BlockSpec((1, 128), lambda j: (_I0, _I0)),
        out_shape=jax.ShapeDtypeStruct((1, 128), jnp.float32),
        name="edge_min",
        scratch_shapes=[pltpu.VMEM((1, 128), jnp.float32)],
    )(ap)


def _w32_body(ap_ref, m_ref, we_ref, sel_ref, out_ref):
    p = ap_ref[...] - m_ref[...]
    q = jnp.exp(-(p * p) / SIGMA2) * we_ref[...]
    sv = jnp.dot(q, sel_ref[...], preferred_element_type=jnp.float32)
    out_ref[...] = 1.0 / (1.0 + jnp.exp(-sv))


def _w32(ap, m128, we128, sel):
    return pl.pallas_call(
        _w32_body,
        grid=(NB_P,),
        in_specs=[
            pl.BlockSpec((BR, 128), lambda j: (j, _I0)),
            pl.BlockSpec((1, 128), lambda j: (_I0, _I0)),
            pl.BlockSpec((1, 128), lambda j: (_I0, _I0)),
            pl.BlockSpec((128, 32), lambda j: (_I0, _I0)),
        ],
        out_specs=pl.BlockSpec((BR, 32), lambda j: (j, _I0)),
        out_shape=jax.ShapeDtypeStruct((EP32, 32), jnp.float32),
        name="edge_w32",
    )(ap, m128, we128, sel)


def _rep_body(wf_ref, rr_ref, out_ref):
    out_ref[...] = jnp.dot(wf_ref[...], rr_ref[...],
                           preferred_element_type=jnp.float32)


def _replicate(wf, rr):
    return pl.pallas_call(
        _rep_body,
        grid=(NB_R,),
        in_specs=[
            pl.BlockSpec((BR8, 8), lambda j: (j, _I0)),
            pl.BlockSpec((8, 128), lambda j: (_I0, _I0)),
        ],
        out_specs=pl.BlockSpec((BR8, 128), lambda j: (j, _I0)),
        out_shape=jax.ShapeDtypeStruct((ER8, 128), jnp.float32),
        name="edge_w_rep",
    )(wf, rr)


def _edge_weights(edge_attr, We):
    ap = edge_attr.reshape(EP32, 128)
    lane_min = _lane_min(ap)
    min4 = lane_min.reshape(32, D_EDGE).min(axis=0)
    m128 = jnp.tile(min4, 32).reshape(1, 128)
    we128 = jnp.tile(We.reshape(D_EDGE), 32).reshape(1, 128)
    sel = jnp.repeat(jnp.eye(32, dtype=jnp.float32), D_EDGE, axis=0)
    w32 = _w32(ap, m128, we128, sel)
    rr = jnp.repeat(jnp.eye(8, dtype=jnp.float32), WREP, axis=1)
    wrep = _replicate(w32.reshape(ER8, 8), rr)
    return wrep.reshape(E, WREP)


BN = 1000
NB_H = N // BN


def _h_body(x_ref, w1_ref, out_ref):
    out_ref[...] = jnp.dot(x_ref[...], w1_ref[...],
                           preferred_element_type=jnp.float32)


def _node_emb(x_real, W1):
    return pl.pallas_call(
        _h_body,
        grid=(NB_H,),
        in_specs=[
            pl.BlockSpec((BN, D_FEAT), lambda j: (j, _I0)),
            pl.BlockSpec((D_FEAT, D_EMB), lambda j: (_I0, _I0)),
        ],
        out_specs=pl.BlockSpec((BN, D_EMB), lambda j: (j, _I0)),
        out_shape=jax.ShapeDtypeStruct((N, D_EMB), jnp.float32),
        name="h_mm",
    )(x_real, W1)


NTILES = 32
EPT = E // NTILES
C = 125
K = EPT // C
PAIRS = K // 2
RPT = N // 16


def _sc_body(h_hbm, src_hbm, dst_hbm, w_hbm, out_hbm,
             src_v, dst0, dst1, w0, w1, rows0, rows1, agg_sh, sem0, sem1):
    cid = lax.axis_index("c")
    sid = lax.axis_index("s")
    wid = cid * 16 + sid

    def _zb(i, carry):
        for k in range(8):
            rows0[i, pl.ds(k * 16, 16)] = jnp.zeros((16,), jnp.float32)
        return carry
    lax.fori_loop(jnp.int32(0), jnp.int32(C), _zb, jnp.int32(0))
    for t in range(5):
        pltpu.sync_copy(rows0, agg_sh.at[pl.ds(sid * RPT + t * C, C)])
    plsc.subcore_barrier()

    pltpu.sync_copy(src_hbm.at[wid], src_v)

    def _issue(j, rows, dst, w, sem):
        pltpu.async_copy(h_hbm.at[src_v.at[j]], rows, sem)
        pltpu.async_copy(dst_hbm.at[wid, j], dst, sem)
        pltpu.async_copy(w_hbm.at[wid, j], w, sem)

    def _wait(j, rows, dst, w, sem):
        pltpu.make_async_copy(h_hbm.at[src_v.at[j]], rows, sem).wait()
        pltpu.make_async_copy(dst_hbm.at[wid, j], dst, sem).wait()
        pltpu.make_async_copy(w_hbm.at[wid, j], w, sem).wait()

    def _scale(rows, w):
        def body(e, c2):
            off = pl.multiple_of(e * WREP, WREP)
            wj = w[pl.ds(off, WREP)]
            for k in range(8):
                sl = pl.ds(k * 16, 16)
                rows[e, sl] = rows[e, sl] * wj
            return c2
        lax.fori_loop(jnp.int32(0), jnp.int32(C), body, jnp.int32(0))

    _issue(jnp.int32(0), rows0, dst0, w0, sem0)
    _wait(jnp.int32(0), rows0, dst0, w0, sem0)

    def _pair(m, carry):
        a = 2 * m
        b = a + 1

        _issue(b, rows1, dst1, w1, sem1)
        _scale(rows0, w0)
        _wait(b, rows1, dst1, w1, sem1)
        pltpu.sync_copy(rows0, agg_sh.at[dst0], add=True)

        @pl.when(m < PAIRS - 1)
        def _():
            _issue(a + 2, rows0, dst0, w0, sem0)
        _scale(rows1, w1)

        @pl.when(m < PAIRS - 1)
        def _():
            _wait(a + 2, rows0, dst0, w0, sem0)
        pltpu.sync_copy(rows1, agg_sh.at[dst1], add=True)
        return carry
    lax.fori_loop(jnp.int32(0), jnp.int32(PAIRS), _pair, jnp.int32(0))
    plsc.subcore_barrier()

    pltpu.sync_copy(agg_sh.at[pl.ds(sid * RPT, RPT)], out_hbm.at[wid])


def _sc_scatter(h, src3d, dst3d, w3):
    mesh = plsc.VectorSubcoreMesh(core_axis_name="c", subcore_axis_name="s")
    fn = functools.partial(
        pl.kernel,
        out_type=jax.ShapeDtypeStruct((NTILES, RPT, D_EMB), jnp.float32),
        mesh=mesh,
        name="sc_scatter",
        scratch_types=[
            pltpu.VMEM((K, C), jnp.int32),
            pltpu.VMEM((C,), jnp.int32),
            pltpu.VMEM((C,), jnp.int32),
            pltpu.VMEM((C * WREP,), jnp.float32),
            pltpu.VMEM((C * WREP,), jnp.float32),
            pltpu.VMEM((C, D_EMB), jnp.float32),
            pltpu.VMEM((C, D_EMB), jnp.float32),
            pltpu.VMEM_SHARED((N, D_EMB), jnp.float32),
            pltpu.SemaphoreType.DMA,
            pltpu.SemaphoreType.DMA,
        ],
    )(_sc_body)
    return fn(h, src3d, dst3d, w3)


def _final_body(p0_ref, p1_ref, w2_ref, b2_ref, wc_ref, bc_ref, g_ref,
                out_ref, acc_ref):
    j = pl.program_id(0)

    @pl.when(j == 0)
    def _():
        acc_ref[...] = jnp.zeros_like(acc_ref)
        out_ref[...] = jnp.zeros_like(out_ref)

    s = p0_ref[...] + p1_ref[...]
    t = jnp.dot(s, w2_ref[...], preferred_element_type=jnp.float32)
    t = jnp.maximum(t + b2_ref[...], 0.0)
    acc_ref[...] += jnp.sum(t, axis=0, keepdims=True)

    @pl.when(j == NB_H - 1)
    def _():
        o = jnp.dot(acc_ref[...], wc_ref[pl.ds(0, D_EMB), :],
                    preferred_element_type=jnp.float32)
        o += jnp.dot(g_ref[...], wc_ref[pl.ds(D_EMB, D_GLOBAL), :],
                     preferred_element_type=jnp.float32)
        out_ref[...] = o + bc_ref[...]


def _final(parts, W2, b2r, Wc, bcr, g):
    return pl.pallas_call(
        _final_body,
        grid=(NB_H,),
        in_specs=[
            pl.BlockSpec((BN, D_EMB), lambda j: (j, _I0)),
            pl.BlockSpec((BN, D_EMB), lambda j: (j + NB_H, _I0)),
            pl.BlockSpec((D_EMB, D_EMB), lambda j: (_I0, _I0)),
            pl.BlockSpec((1, D_EMB), lambda j: (_I0, _I0)),
            pl.BlockSpec((D_EMB + D_GLOBAL, N_CLASSES), lambda j: (_I0, _I0)),
            pl.BlockSpec((1, N_CLASSES), lambda j: (_I0, _I0)),
            pl.BlockSpec((1, D_GLOBAL), lambda j: (_I0, _I0)),
        ],
        out_specs=pl.BlockSpec((1, N_CLASSES), lambda j: (_I0, _I0)),
        out_shape=jax.ShapeDtypeStruct((1, N_CLASSES), jnp.float32),
        name="final",
        scratch_shapes=[pltpu.VMEM((1, D_EMB), jnp.float32)],
    )(parts, parts, W2, b2r, Wc, bcr, g)


def kernel(x, edge_index, edge_attr, batch, W1, We, W2, b2, Wc, bc):
    src = edge_index[0].astype(jnp.int32)
    dst = edge_index[1].astype(jnp.int32)
    src3d = src.reshape(NTILES, K, C)
    dst3d = dst.reshape(NTILES, K, C)
    x_real = x[:, :D_FEAT]
    g = x[0:1, D_FEAT:]

    w_e = _edge_weights(edge_attr, We)
    w3 = w_e.reshape(NTILES, K, C * WREP)
    h = _node_emb(x_real, W1)
    parts = _sc_scatter(h, src3d, dst3d, w3)
    parts = parts.reshape(2 * N, D_EMB)
    out = _final(parts, W2, b2.reshape(1, D_EMB), Wc,
                 bc.reshape(1, N_CLASSES), g)
    return out
